# baseline (speedup 1.0000x reference)
import functools

import jax
import jax.numpy as jnp
from jax import lax
from jax.experimental import pallas as pl
from jax.experimental.pallas import tpu as pltpu
from jax.experimental.pallas import tpu_sc as plsc

_B, _N, _D = 4, 8192, 1024
_NC, _NS = 2, 16
_NW = _NC * _NS
_RPW = _N // _NW
_C = 8
_CHUNKS = _RPW // _C
_CW = _C * _D
_L = 16
_STEPS = _CHUNKS * _B
_NXB = 8
_PREF = 4


def _sc_body(x_hbm, pe_hbm, out_hbm, *refs):
    xbs = list(refs[:_NXB])
    pebs = list(refs[_NXB:_NXB + 2])
    in_sems = list(refs[_NXB + 2:2 * _NXB + 2])
    out_sems = list(refs[2 * _NXB + 2:3 * _NXB + 2])
    pe_sems = list(refs[3 * _NXB + 2:3 * _NXB + 4])

    wid = lax.axis_index("s") * _NC + lax.axis_index("c")
    n_base = wid * _RPW

    def x_slice(t):
        i, b = divmod(t, _B)
        return pl.ds((b * _N + n_base + i * _C) * _D, _CW)

    def start_in(t):
        s = t % _NXB
        h = pltpu.make_async_copy(x_hbm.at[x_slice(t)], xbs[s], in_sems[s])
        h.start()
        return h

    def start_out(t):
        s = t % _NXB
        h = pltpu.make_async_copy(xbs[s], out_hbm.at[x_slice(t)], out_sems[s])
        h.start()
        return h

    def start_pe(i):
        p = i % 2
        h = pltpu.make_async_copy(
            pe_hbm.at[pl.ds((n_base + i * _C) * _D, _CW)], pebs[p], pe_sems[p])
        h.start()
        return h

    in_h = [None] * _STEPS
    out_h = [None] * _STEPS
    pe_h = [None] * _CHUNKS

    pe_h[0] = start_pe(0)
    for t in range(_PREF):
        in_h[t] = start_in(t)

    for t in range(_STEPS):
        i, b = divmod(t, _B)
        if t + _PREF < _STEPS:
            if t + _PREF >= _NXB:
                out_h[t + _PREF - _NXB].wait()
            in_h[t + _PREF] = start_in(t + _PREF)
        if b == 0:
            pe_h[i].wait()
            if i + 1 < _CHUNKS:
                pe_h[i + 1] = start_pe(i + 1)
        in_h[t].wait()

        xb = xbs[t % _NXB]
        peb = pebs[i % 2]

        @pl.loop(0, _CW // _L, unroll=8)
        def _add(k):
            off = k * _L
            plsc.addupdate(xb.at[pl.ds(off, _L)], peb[pl.ds(off, _L)])

        out_h[t] = start_out(t)

    for t in range(_STEPS - _NXB, _STEPS):
        out_h[t].wait()


_sc_add = functools.partial(
    pl.kernel,
    out_type=jax.ShapeDtypeStruct((_B * _N * _D,), jnp.float32),
    mesh=plsc.VectorSubcoreMesh(
        core_axis_name="c", subcore_axis_name="s",
        num_cores=_NC, num_subcores=_NS,
    ),
    scratch_types=(
        [pltpu.VMEM((_CW,), jnp.float32) for _ in range(_NXB + 2)]
        + [pltpu.SemaphoreType.DMA for _ in range(2 * _NXB + 2)]
    ),
)(_sc_body)


def kernel(x, pos_emb):
    B, N, D = x.shape
    xf = x.reshape(B * N * D)
    pef = pos_emb.reshape(-1)[: N * D]
    out = _sc_add(xf, pef)
    return out.reshape(B, N, D)


# speedup vs baseline: 4.1837x; 4.1837x over previous
import jax
import jax.numpy as jnp
from jax.experimental import pallas as pl


_TN = 256


def _add_block(x_ref, pe_ref, o_ref):
    o_ref[...] = x_ref[...] + pe_ref[...]


def kernel(x, pos_emb):
    B, N, D = x.shape
    pe = pos_emb[:N]
    return pl.pallas_call(
        _add_block,
        grid=(N // _TN,),
        in_specs=[
            pl.BlockSpec((B, _TN, D), lambda n: (0, n, 0)),
            pl.BlockSpec((_TN, D), lambda n: (n, 0)),
        ],
        out_specs=pl.BlockSpec((B, _TN, D), lambda n: (0, n, 0)),
        out_shape=jax.ShapeDtypeStruct((B, N, D), x.dtype),
    )(x, pe)
